# baseline (speedup 1.0000x reference)
import functools

import jax
import jax.numpy as jnp
from jax import lax
from jax.experimental import pallas as pl
from jax.experimental.pallas import tpu as pltpu
from jax.experimental.pallas import tpu_sc as plsc

N_TIME = 8192
BINS = 10000
OUT_WIDTH = 2048
BATCH = 4

_TOTAL_ROWS = BATCH * N_TIME
_SEG = N_TIME // 32
_CHUNK = 16
_BLK = 8
_NRING = 6
_MAX_BLK_START = BINS - _BLK


def _build_sc_call():
    info = plsc.get_sparse_core_info()
    nc, ns, nl = info.num_cores, info.num_subcores, info.num_lanes
    nw = nc * ns

    mesh = plsc.VectorSubcoreMesh(core_axis_name="c", subcore_axis_name="s")

    n_chunks = BATCH * (_SEG // _CHUNK)
    chunks_per_seg = _SEG // _CHUNK

    @functools.partial(
        pl.kernel,
        mesh=mesh,
        out_type=jax.ShapeDtypeStruct((_TOTAL_ROWS, 1, OUT_WIDTH), jnp.float32),
        scratch_types=[
            pltpu.VMEM((BATCH, 2, 16), jnp.float32),
            pltpu.VMEM((BATCH * _SEG,), jnp.int32),
            pltpu.VMEM((_NRING * _BLK, 1, OUT_WIDTH), jnp.float32),
            pltpu.SemaphoreType.DMA,
            pltpu.SemaphoreType.DMA,
        ],
    )
    def sc_kernel(params_hbm, table_hbm, out_hbm, params_v, idx_v, ring_v,
                  fsem, wsem):
        wid = lax.axis_index("s") * nc + lax.axis_index("c")
        t_base = wid * _SEG

        pltpu.sync_copy(params_hbm, params_v)

        lane = lax.iota(jnp.int32, nl).astype(jnp.float32)
        tb_f = t_base.astype(jnp.float32)

        for b in range(BATCH):
            sv = params_v[b, 0, :]
            dv = params_v[b, 1, :]

            def idx_body(i, _, sv=sv, dv=dv, b=b):
                tv = tb_f + (i * nl).astype(jnp.float32) + lane
                pos = sv + dv * (tv * (1.0 / N_TIME))
                idx_v[pl.ds(b * _SEG + i * nl, nl)] = (
                    pos * float(BINS)).astype(jnp.int32)
                return 0

            lax.fori_loop(0, _SEG // nl, idx_body, 0)

        def row_wait():
            pltpu.make_async_copy(
                ring_v.at[pl.ds(0, 1)],
                out_hbm.at[pl.ds(t_base, 1)],
                wsem,
            ).wait()

        def chunk_body(c, carry):
            first_idx, fetched_n = carry
            seg = c // chunks_per_seg
            ck = c - seg * chunks_per_seg
            seg_lo = seg * _SEG

            ends_first = idx_v[pl.ds(seg_lo, nl)]
            ends_last = idx_v[pl.ds(seg_lo + _SEG - nl, nl)]
            i_first = ends_first[0]
            i_last = ends_last[nl - 1]
            is_new_seg = ck == 0
            rev = i_first > i_last
            first_idx = jnp.where(is_new_seg,
                                  jnp.minimum(i_first, i_last), first_idx)
            fetched_n = jnp.where(is_new_seg, jnp.int32(-1), fetched_n)

            fwd_base = seg_lo + ck * _CHUNK
            rev_base = seg_lo + (_SEG - _CHUNK) - ck * _CHUNK
            vec_f = idx_v[pl.ds(fwd_base, _CHUNK)]
            vec_r = idx_v[pl.ds(rev_base, _CHUNK)]

            def row_of(r):
                ridx = jnp.where(rev, vec_r[_CHUNK - 1 - r], vec_f[r])
                k = jnp.where(rev,
                              jnp.int32(_SEG - 1) - (ck * _CHUNK + r),
                              ck * _CHUNK + r)
                return ridx, k

            @pl.when(c > 0)
            def _():
                for _r in range(_CHUNK):
                    row_wait()

            idx_hi, _ = row_of(_CHUNK - 1)
            n_hi = lax.shift_right_logical(idx_hi - first_idx, 3)

            for _f in range(4):
                need = fetched_n < n_hi

                @pl.when(need)
                def _(fetched_n=fetched_n):
                    nn = fetched_n + 1
                    blk_start = jnp.minimum(first_idx + nn * _BLK,
                                            jnp.int32(_MAX_BLK_START))
                    slot = lax.rem(nn, jnp.int32(_NRING))
                    fd = pltpu.make_async_copy(
                        table_hbm.at[pl.ds(blk_start, _BLK)],
                        ring_v.at[pl.ds(slot * _BLK, _BLK)],
                        fsem,
                    )
                    fd.start()
                    fd.wait()

                fetched_n = jnp.where(need, fetched_n + 1, fetched_n)

            for r in range(_CHUNK):
                row_idx, k = row_of(r)
                n_k = lax.shift_right_logical(row_idx - first_idx, 3)
                blk_start = jnp.minimum(first_idx + n_k * _BLK,
                                        jnp.int32(_MAX_BLK_START))
                slot = lax.rem(n_k, jnp.int32(_NRING))
                off = row_idx - blk_start
                out_r = seg * N_TIME + t_base + k
                pltpu.make_async_copy(
                    ring_v.at[pl.ds(slot * _BLK + off, 1)],
                    out_hbm.at[pl.ds(out_r, 1)],
                    wsem,
                ).start()

            return (first_idx, fetched_n)

        lax.fori_loop(0, n_chunks, chunk_body,
                      (jnp.int32(0), jnp.int32(-1)))

        for _r in range(_CHUNK):
            row_wait()

    return sc_kernel


def kernel(pos_start, pos_end, emb_weight):
    s = pos_start.reshape(BATCH)
    d = pos_end.reshape(BATCH) - s
    params = jnp.stack([s, d], axis=1)
    params = jnp.broadcast_to(params[:, :, None], (BATCH, 2, 16))
    sc_call = _build_sc_call()
    out = sc_call(params, emb_weight.reshape(BINS, 1, OUT_WIDTH))
    return out.reshape(BATCH, N_TIME, OUT_WIDTH)


# speedup vs baseline: 1.0815x; 1.0815x over previous
import functools

import jax
import jax.numpy as jnp
from jax import lax
from jax.experimental import pallas as pl
from jax.experimental.pallas import tpu as pltpu
from jax.experimental.pallas import tpu_sc as plsc

N_TIME = 8192
BINS = 10000
OUT_WIDTH = 2048
BATCH = 4

_TOTAL_ROWS = BATCH * N_TIME
_SEG = N_TIME // 32
_CHUNK = 16
_BLK = 8
_NRING = 6
_MAX_BLK_START = BINS - _BLK


def _build_sc_call():
    info = plsc.get_sparse_core_info()
    nc, ns, nl = info.num_cores, info.num_subcores, info.num_lanes
    nw = nc * ns

    mesh = plsc.VectorSubcoreMesh(core_axis_name="c", subcore_axis_name="s")

    n_chunks = BATCH * (_SEG // _CHUNK)
    chunks_per_seg = _SEG // _CHUNK

    @functools.partial(
        pl.kernel,
        mesh=mesh,
        out_type=jax.ShapeDtypeStruct((_TOTAL_ROWS, 1, OUT_WIDTH), jnp.float32),
        scratch_types=[
            pltpu.VMEM((BATCH, 2, 16), jnp.float32),
            pltpu.VMEM((BATCH * _SEG,), jnp.int32),
            pltpu.VMEM((_NRING * _BLK, 1, OUT_WIDTH), jnp.float32),
            pltpu.SemaphoreType.DMA,
            pltpu.SemaphoreType.DMA,
        ],
    )
    def sc_kernel(params_hbm, table_hbm, out_hbm, params_v, idx_v, ring_v,
                  fsem, wsem):
        wid = lax.axis_index("s") * nc + lax.axis_index("c")
        t_base = wid * _SEG

        pltpu.sync_copy(params_hbm, params_v)

        lane = lax.iota(jnp.int32, nl).astype(jnp.float32)
        tb_f = t_base.astype(jnp.float32)

        for b in range(BATCH):
            sv = params_v[b, 0, :]
            dv = params_v[b, 1, :]

            def idx_body(i, _, sv=sv, dv=dv, b=b):
                tv = tb_f + (i * nl).astype(jnp.float32) + lane
                pos = sv + dv * (tv * (1.0 / N_TIME))
                idx_v[pl.ds(b * _SEG + i * nl, nl)] = (
                    pos * float(BINS)).astype(jnp.int32)
                return 0

            lax.fori_loop(0, _SEG // nl, idx_body, 0)

        def row_wait():
            pltpu.make_async_copy(
                ring_v.at[pl.ds(0, 1)],
                out_hbm.at[pl.ds(t_base, 1)],
                wsem,
            ).wait()

        def fetch_wait():
            pltpu.make_async_copy(
                table_hbm.at[pl.ds(0, _BLK)],
                ring_v.at[pl.ds(0, _BLK)],
                fsem,
            ).wait()

        def chunk_body(c, carry):
            first_idx, started_n, waited_n = carry
            seg = c // chunks_per_seg
            ck = c - seg * chunks_per_seg
            seg_lo = seg * _SEG

            ends_first = idx_v[pl.ds(seg_lo, nl)]
            ends_last = idx_v[pl.ds(seg_lo + _SEG - nl, nl)]
            i_first = ends_first[0]
            i_last = ends_last[nl - 1]
            is_new_seg = ck == 0
            rev = i_first > i_last
            first_idx = jnp.where(is_new_seg,
                                  jnp.minimum(i_first, i_last), first_idx)
            started_n = jnp.where(is_new_seg, jnp.int32(-1), started_n)
            waited_n = jnp.where(is_new_seg, jnp.int32(-1), waited_n)

            fwd_base = seg_lo + ck * _CHUNK
            rev_base = seg_lo + (_SEG - _CHUNK) - ck * _CHUNK
            chunk_base = jnp.where(rev, rev_base, fwd_base)
            vec = idx_v[pl.ds(chunk_base, _CHUNK)]

            idx_hi = jnp.maximum(vec[0], vec[_CHUNK - 1])
            ck_n = jnp.minimum(ck + 1, jnp.int32(chunks_per_seg - 1))
            nxt_base = jnp.where(rev,
                                 seg_lo + (_SEG - _CHUNK) - ck_n * _CHUNK,
                                 seg_lo + ck_n * _CHUNK)
            vec_n = idx_v[pl.ds(nxt_base, _CHUNK)]
            idx_hi_n = jnp.maximum(
                jnp.maximum(vec_n[0], vec_n[_CHUNK - 1]), idx_hi)
            n_hi = lax.shift_right_logical(idx_hi - first_idx, 3)
            n_hi_n = lax.shift_right_logical(idx_hi_n - first_idx, 3)

            @pl.when(c > 0)
            def _():
                for _r in range(_CHUNK):
                    row_wait()

            for _f in range(6):
                need = started_n < n_hi_n

                @pl.when(need)
                def _(started_n=started_n):
                    nn = started_n + 1
                    blk_start = jnp.minimum(first_idx + nn * _BLK,
                                            jnp.int32(_MAX_BLK_START))
                    slot = lax.rem(nn, jnp.int32(_NRING))
                    pltpu.make_async_copy(
                        table_hbm.at[pl.ds(blk_start, _BLK)],
                        ring_v.at[pl.ds(slot * _BLK, _BLK)],
                        fsem,
                    ).start()

                started_n = jnp.where(need, started_n + 1, started_n)

            for _f in range(6):
                need = waited_n < n_hi

                @pl.when(need)
                def _():
                    fetch_wait()

                waited_n = jnp.where(need, waited_n + 1, waited_n)

            out_base = seg * N_TIME + t_base + (chunk_base - seg_lo)
            for r in range(_CHUNK):
                row_idx = vec[r]
                n_k = lax.shift_right_logical(row_idx - first_idx, 3)
                blk_start = jnp.minimum(first_idx + n_k * _BLK,
                                        jnp.int32(_MAX_BLK_START))
                slot = lax.rem(n_k, jnp.int32(_NRING))
                off = row_idx - blk_start
                pltpu.make_async_copy(
                    ring_v.at[pl.ds(slot * _BLK + off, 1)],
                    out_hbm.at[pl.ds(out_base + r, 1)],
                    wsem,
                ).start()

            return (first_idx, started_n, waited_n)

        lax.fori_loop(0, n_chunks, chunk_body,
                      (jnp.int32(0), jnp.int32(-1), jnp.int32(-1)))

        for _r in range(_CHUNK):
            row_wait()

    return sc_kernel


def kernel(pos_start, pos_end, emb_weight):
    s = pos_start.reshape(BATCH)
    d = pos_end.reshape(BATCH) - s
    params = jnp.stack([s, d], axis=1)
    params = jnp.broadcast_to(params[:, :, None], (BATCH, 2, 16))
    sc_call = _build_sc_call()
    out = sc_call(params, emb_weight.reshape(BINS, 1, OUT_WIDTH))
    return out.reshape(BATCH, N_TIME, OUT_WIDTH)


# speedup vs baseline: 3.0790x; 2.8469x over previous
import functools

import jax
import jax.numpy as jnp
from jax import lax
from jax.experimental import pallas as pl
from jax.experimental.pallas import tpu as pltpu
from jax.experimental.pallas import tpu_sc as plsc

N_TIME = 8192
BINS = 10000
OUT_WIDTH = 2048
BATCH = 4

_TOTAL_ROWS = BATCH * N_TIME
_SEG = N_TIME // 32
_CHUNK = 16
_BLK = 8
_NRING = 6


def _build_sc_call():
    info = plsc.get_sparse_core_info()
    nc, ns, nl = info.num_cores, info.num_subcores, info.num_lanes

    mesh = plsc.VectorSubcoreMesh(core_axis_name="c", subcore_axis_name="s")

    n_chunks = BATCH * (_SEG // _CHUNK)
    chunks_per_seg = _SEG // _CHUNK

    @functools.partial(
        pl.kernel,
        mesh=mesh,
        out_type=jax.ShapeDtypeStruct((_TOTAL_ROWS, OUT_WIDTH), jnp.float32),
        scratch_types=[
            pltpu.VMEM((BATCH, 2, 16), jnp.float32),
            pltpu.VMEM((BATCH * _SEG,), jnp.int32),
            pltpu.VMEM((_NRING * _BLK, OUT_WIDTH), jnp.float32),
            pltpu.SemaphoreType.DMA,
            pltpu.SemaphoreType.DMA,
        ],
    )
    def sc_kernel(params_hbm, table_hbm, out_hbm, params_v, idx_v, ring_v,
                  fsem, wsem):
        wid = lax.axis_index("s") * nc + lax.axis_index("c")
        t_base = wid * _SEG

        pltpu.sync_copy(params_hbm, params_v)

        lane = lax.iota(jnp.int32, nl).astype(jnp.float32)
        tb_f = t_base.astype(jnp.float32)

        for b in range(BATCH):
            sv = params_v[b, 0, :]
            dv = params_v[b, 1, :]

            def idx_body(i, _, sv=sv, dv=dv, b=b):
                tv = tb_f + (i * nl).astype(jnp.float32) + lane
                pos = sv + dv * (tv * (1.0 / N_TIME))
                idx_v[pl.ds(b * _SEG + i * nl, nl)] = (
                    pos * float(BINS)).astype(jnp.int32)
                return 0

            lax.fori_loop(0, _SEG // nl, idx_body, 0)

        def row_wait():
            pltpu.make_async_copy(
                ring_v.at[pl.ds(0, 1)],
                out_hbm.at[pl.ds(t_base, 1)],
                wsem,
            ).wait()

        def fetch_wait():
            pltpu.make_async_copy(
                table_hbm.at[pl.ds(0, _BLK)],
                ring_v.at[pl.ds(0, _BLK)],
                fsem,
            ).wait()

        def start_fetch(bn):
            blk_start = pl.multiple_of(bn * _BLK, _BLK)
            slot = lax.rem(bn, jnp.int32(_NRING))
            pltpu.make_async_copy(
                table_hbm.at[pl.ds(blk_start, _BLK)],
                ring_v.at[pl.ds(pl.multiple_of(slot * _BLK, _BLK), _BLK)],
                fsem,
            ).start()

        def chunk_body(c, carry):
            started_bn, waited_bn = carry
            seg = c // chunks_per_seg
            ck = c - seg * chunks_per_seg
            seg_lo = seg * _SEG

            ends_first = idx_v[pl.ds(seg_lo, nl)]
            ends_last = idx_v[pl.ds(seg_lo + _SEG - nl, nl)]
            rev = ends_first[0] > ends_last[nl - 1]
            is_new_seg = ck == 0

            fwd_base = seg_lo + ck * _CHUNK
            rev_base = seg_lo + (_SEG - _CHUNK) - ck * _CHUNK
            chunk_base = jnp.where(rev, rev_base, fwd_base)
            vec = idx_v[pl.ds(chunk_base, _CHUNK)]

            idx_lo = jnp.minimum(vec[0], vec[_CHUNK - 1])
            idx_hi = jnp.maximum(vec[0], vec[_CHUNK - 1])
            ck_n = jnp.minimum(ck + 1, jnp.int32(chunks_per_seg - 1))
            nxt_base = jnp.where(rev,
                                 seg_lo + (_SEG - _CHUNK) - ck_n * _CHUNK,
                                 seg_lo + ck_n * _CHUNK)
            vec_n = idx_v[pl.ds(nxt_base, _CHUNK)]
            idx_hi_n = jnp.maximum(
                jnp.maximum(vec_n[0], vec_n[_CHUNK - 1]), idx_hi)
            bn_lo = lax.shift_right_logical(idx_lo, 3)
            bn_hi = lax.shift_right_logical(idx_hi, 3)
            bn_hi_n = lax.shift_right_logical(idx_hi_n, 3)

            started_bn = jnp.where(is_new_seg, bn_lo - 1, started_bn)
            waited_bn = jnp.where(is_new_seg, bn_lo - 1, waited_bn)

            @pl.when(c > 0)
            def _():
                for _r in range(_CHUNK):
                    row_wait()

            for _f in range(6):
                need = started_bn < bn_hi_n

                @pl.when(need)
                def _(started_bn=started_bn):
                    start_fetch(started_bn + 1)

                started_bn = jnp.where(need, started_bn + 1, started_bn)

            for _f in range(6):
                need = waited_bn < bn_hi

                @pl.when(need)
                def _():
                    fetch_wait()

                waited_bn = jnp.where(need, waited_bn + 1, waited_bn)

            out_base = pl.multiple_of(
                seg * N_TIME + t_base + (chunk_base - seg_lo), _CHUNK)
            for r in range(_CHUNK):
                row_idx = vec[r]
                slot = lax.rem(lax.shift_right_logical(row_idx, 3),
                               jnp.int32(_NRING))
                off = lax.bitwise_and(row_idx, jnp.int32(_BLK - 1))
                pltpu.make_async_copy(
                    ring_v.at[pl.ds(slot * _BLK + off, 1)],
                    out_hbm.at[pl.ds(out_base + r, 1)],
                    wsem,
                ).start()

            return (started_bn, waited_bn)

        lax.fori_loop(0, n_chunks, chunk_body,
                      (jnp.int32(-1), jnp.int32(-1)))

        for _r in range(_CHUNK):
            row_wait()

    return sc_kernel


def kernel(pos_start, pos_end, emb_weight):
    s = pos_start.reshape(BATCH)
    d = pos_end.reshape(BATCH) - s
    params = jnp.stack([s, d], axis=1)
    params = jnp.broadcast_to(params[:, :, None], (BATCH, 2, 16))
    sc_call = _build_sc_call()
    out = sc_call(params, emb_weight)
    return out.reshape(BATCH, N_TIME, OUT_WIDTH)
